# baseline (speedup 1.0000x reference)
import jax
import jax.numpy as jnp
from jax import lax
from jax.experimental import pallas as pl
from jax.experimental.pallas import tpu as pltpu
from jax.experimental.pallas import tpu_sc as plsc

DIV_GUARD = 1e-05

NC, NS, L = 2, 16, 16
NW = NC * NS

B, F, T = 16, 512, 2048
ROWS = B * F
KSC = 1
SC_ROWS = KSC * F
RPW = SC_ROWS // NW
RC = 8
NCHUNK = RPW // RC
TV = T // L
UB = 8
NB = TV // UB


def _lane_shuffle(v, perm):
    dnums = lax.GatherDimensionNumbers(
        offset_dims=(), collapsed_slice_dims=(0,), start_index_map=(0,)
    )
    return lax.gather(
        v, perm[:, None], dnums, (1,),
        mode=lax.GatherScatterMode.PROMISE_IN_BOUNDS,
    )


def _row_normalize(buf, bit_v, r, n_i, n_f, fb, lanes, zeros):
    def p1(jb, carry):
        s, ss = carry
        for u in range(UB):
            v = buf[r, pl.ds((jb * UB + u) * L, L)]
            s = s + v
            ss = ss + v * v
        return s, ss

    s, ss = lax.fori_loop(0, fb, p1, (zeros, zeros))
    for u in range(UB):
        j = fb * UB + u
        t = lanes + j * L
        v = buf[r, pl.ds(j * L, L)]
        vm = jnp.where(t < n_i, v, 0.0)
        s = s + vm
        ss = ss + vm * vm
    for sh in (8, 4, 2, 1):
        perm = lanes ^ sh
        s = s + _lane_shuffle(s, perm)
        ss = ss + _lane_shuffle(ss, perm)
    mean_v = s / n_f
    var_v = (ss - n_f * mean_v * mean_v) / (n_f - 1.0)
    var_v = jnp.maximum(var_v, 1e-30)
    bit_v.bitcast(jnp.float32)[0, :] = var_v
    iv = bit_v[0, :]
    iv = 0x5F3759DF - lax.shift_right_logical(iv, 1)
    bit_v[0, :] = iv
    y = bit_v.bitcast(jnp.float32)[0, :]
    for _ in range(3):
        y = y * (1.5 - 0.5 * var_v * y * y)
    std = var_v * y + DIV_GUARD
    inv = 1.0 / std

    def p2(jb, _):
        for u in range(UB):
            j2 = jb * UB + u
            v = buf[r, pl.ds(j2 * L, L)]
            buf[r, pl.ds(j2 * L, L)] = (v - mean_v) * inv
        return 0

    lax.fori_loop(0, fb, p2, 0)
    for u in range(UB):
        j = fb * UB + u
        t = lanes + j * L
        v = buf[r, pl.ds(j * L, L)]
        buf[r, pl.ds(j * L, L)] = jnp.where(t < n_i, (v - mean_v) * inv, 0.0)

    def p3(jb, _):
        for u in range(UB):
            buf[r, pl.ds((jb * UB + u) * L, L)] = zeros
        return 0

    lax.fori_loop(fb + 1, NB, p3, 0)


def _sc_body(x_hbm, sl_hbm, out_hbm, sl_v, bit_v, buf):
    wid = lax.axis_index("s") * NC + lax.axis_index("c")
    b = wid // (NW // KSC)
    pltpu.sync_copy(sl_hbm, sl_v)
    lanes = lax.iota(jnp.int32, L)
    zeros = jnp.zeros((L,), jnp.float32)
    slv = sl_v[...]
    n_i = jnp.int32(0)
    for j in range(L):
        n_i = jnp.where(b == j, slv[j], n_i)
    n_f = n_i.astype(jnp.float32)
    fb = n_i // (UB * L)
    base = wid * RPW

    def chunk_body(c, _):
        row0 = base + c * RC
        pltpu.sync_copy(x_hbm.at[pl.ds(row0, RC)], buf)

        def rows(r, _2):
            _row_normalize(buf, bit_v, r, n_i, n_f, fb, lanes, zeros)
            return 0

        lax.fori_loop(0, RC, rows, 0)
        pltpu.sync_copy(buf, out_hbm.at[pl.ds(row0, RC)])
        return 0

    lax.fori_loop(0, NCHUNK, chunk_body, 0)


def _sc_part(x2, sl):
    mesh = plsc.VectorSubcoreMesh(
        core_axis_name="c", subcore_axis_name="s", num_cores=NC, num_subcores=NS
    )
    return pl.kernel(
        _sc_body,
        out_type=jax.ShapeDtypeStruct((ROWS, T), jnp.float32),
        mesh=mesh,
        scratch_types=[
            pltpu.VMEM((L,), jnp.int32),
            pltpu.VMEM((1, L), jnp.int32),
            pltpu.VMEM((RC, T), jnp.float32),
        ],
    )(x2, sl)


FSPLIT = 2
FB = F // FSPLIT


def _tc_body(sl_ref, x_ref, sc_ref, o_ref):
    del sc_ref
    bt = pl.program_id(0)
    n = sl_ref[bt + KSC].astype(jnp.float32)
    xv = x_ref[...]
    t = jax.lax.broadcasted_iota(jnp.int32, (1, 1, T), 2)
    mask = (t < sl_ref[bt + KSC]).astype(jnp.float32)
    xm = xv * mask
    s = jnp.sum(xm, axis=2, keepdims=True)
    ss = jnp.sum(xm * xm, axis=2, keepdims=True)
    mean = s / n
    var = (ss - n * mean * mean) / (n - 1.0)
    var = jnp.maximum(var, 0.0)
    std = jnp.sqrt(var) + DIV_GUARD
    o_ref[...] = (xm - mean * mask) / std


def _tc_part(sl, x, sc_full):
    nb = B - KSC
    return pl.pallas_call(
        _tc_body,
        grid=(nb, FSPLIT),
        in_specs=[
            pl.BlockSpec(memory_space=pltpu.SMEM),
            pl.BlockSpec((1, FB, T), lambda bb, fbl: (bb + KSC, fbl, 0)),
            pl.BlockSpec(memory_space=pl.ANY),
        ],
        out_specs=pl.BlockSpec((1, FB, T), lambda bb, fbl: (bb + KSC, fbl, 0)),
        out_shape=jax.ShapeDtypeStruct((B, F, T), x.dtype),
        input_output_aliases={2: 0},
    )(sl, x, sc_full)


def kernel(x, seq_lens):
    sl = seq_lens.astype(jnp.int32)
    x2 = x.reshape(ROWS, T)
    sc_full = _sc_part(x2, sl).reshape(B, F, T)
    return _tc_part(sl, x, sc_full)


# speedup vs baseline: 1.0997x; 1.0997x over previous
import jax
import jax.numpy as jnp
from jax import lax
from jax.experimental import pallas as pl
from jax.experimental.pallas import tpu as pltpu
from jax.experimental.pallas import tpu_sc as plsc

DIV_GUARD = 1e-05

NC, NS, L = 2, 16, 16
NW = NC * NS

B, F, T = 16, 512, 2048
ROWS = B * F
KSC = 1
SC_ROWS = KSC * F
RPW = SC_ROWS // NW
RC = 8
NCHUNK = RPW // RC
TV = T // L
UB = 8
NB = TV // UB


def _lane_shuffle(v, perm):
    dnums = lax.GatherDimensionNumbers(
        offset_dims=(), collapsed_slice_dims=(0,), start_index_map=(0,)
    )
    return lax.gather(
        v, perm[:, None], dnums, (1,),
        mode=lax.GatherScatterMode.PROMISE_IN_BOUNDS,
    )


def _row_normalize(buf, bit_v, r, n_i, n_f, fb, lanes, zeros):
    def p1(jb, carry):
        s, ss = carry
        for u in range(UB):
            v = buf[r, pl.ds((jb * UB + u) * L, L)]
            s = s + v
            ss = ss + v * v
        return s, ss

    s, ss = lax.fori_loop(0, fb, p1, (zeros, zeros))
    for u in range(UB):
        j = fb * UB + u
        t = lanes + j * L
        v = buf[r, pl.ds(j * L, L)]
        vm = jnp.where(t < n_i, v, 0.0)
        s = s + vm
        ss = ss + vm * vm
    for sh in (8, 4, 2, 1):
        perm = lanes ^ sh
        s = s + _lane_shuffle(s, perm)
        ss = ss + _lane_shuffle(ss, perm)
    mean_v = s / n_f
    var_v = (ss - n_f * mean_v * mean_v) / (n_f - 1.0)
    var_v = jnp.maximum(var_v, 1e-30)
    bit_v.bitcast(jnp.float32)[0, :] = var_v
    iv = bit_v[0, :]
    iv = 0x5F3759DF - lax.shift_right_logical(iv, 1)
    bit_v[0, :] = iv
    y = bit_v.bitcast(jnp.float32)[0, :]
    for _ in range(3):
        y = y * (1.5 - 0.5 * var_v * y * y)
    std = var_v * y + DIV_GUARD
    inv = 1.0 / std

    def p2(jb, _):
        for u in range(UB):
            j2 = jb * UB + u
            v = buf[r, pl.ds(j2 * L, L)]
            buf[r, pl.ds(j2 * L, L)] = (v - mean_v) * inv
        return 0

    lax.fori_loop(0, fb, p2, 0)
    for u in range(UB):
        j = fb * UB + u
        t = lanes + j * L
        v = buf[r, pl.ds(j * L, L)]
        buf[r, pl.ds(j * L, L)] = jnp.where(t < n_i, (v - mean_v) * inv, 0.0)

    def p3(jb, _):
        for u in range(UB):
            buf[r, pl.ds((jb * UB + u) * L, L)] = zeros
        return 0

    lax.fori_loop(fb + 1, NB, p3, 0)


def _sc_body(x_hbm, sl_hbm, out_hbm, sl_v, bit_v, buf):
    wid = lax.axis_index("s") * NC + lax.axis_index("c")
    b = wid // (NW // KSC)
    pltpu.sync_copy(sl_hbm, sl_v)
    lanes = lax.iota(jnp.int32, L)
    zeros = jnp.zeros((L,), jnp.float32)
    slv = sl_v[...]
    n_i = jnp.int32(0)
    for j in range(L):
        n_i = jnp.where(b == j, slv[j], n_i)
    n_f = n_i.astype(jnp.float32)
    fb = n_i // (UB * L)
    base = wid * RPW

    def chunk_body(c, _):
        row0 = base + c * RC
        pltpu.sync_copy(x_hbm.at[pl.ds(row0, RC)], buf)

        def rows(r, _2):
            _row_normalize(buf, bit_v, r, n_i, n_f, fb, lanes, zeros)
            return 0

        lax.fori_loop(0, RC, rows, 0)
        pltpu.sync_copy(buf, out_hbm.at[pl.ds(row0, RC)])
        return 0

    lax.fori_loop(0, NCHUNK, chunk_body, 0)


def _sc_part(x2, sl):
    mesh = plsc.VectorSubcoreMesh(
        core_axis_name="c", subcore_axis_name="s", num_cores=NC, num_subcores=NS
    )
    return pl.kernel(
        _sc_body,
        out_type=jax.ShapeDtypeStruct((ROWS, T), jnp.float32),
        mesh=mesh,
        scratch_types=[
            pltpu.VMEM((L,), jnp.int32),
            pltpu.VMEM((1, L), jnp.int32),
            pltpu.VMEM((RC, T), jnp.float32),
        ],
    )(x2, sl)


def _tc_body(sl_ref, x_ref, sc_ref, o_ref):
    del sc_ref
    bt = pl.program_id(0)
    n = sl_ref[bt + KSC].astype(jnp.float32)
    xv = x_ref[...]
    t = jax.lax.broadcasted_iota(jnp.int32, (1, 1, T), 2)
    mask = (t < sl_ref[bt + KSC]).astype(jnp.float32)
    xm = xv * mask
    s = jnp.sum(xm, axis=2, keepdims=True)
    ss = jnp.sum(xm * xm, axis=2, keepdims=True)
    mean = s / n
    var = (ss - n * mean * mean) / (n - 1.0)
    var = jnp.maximum(var, 0.0)
    std = jnp.sqrt(var) + DIV_GUARD
    o_ref[...] = (xm - mean * mask) / std


def _tc_part(sl, x, sc_full):
    nb = B - KSC
    return pl.pallas_call(
        _tc_body,
        grid=(nb,),
        in_specs=[
            pl.BlockSpec(memory_space=pltpu.SMEM),
            pl.BlockSpec((1, F, T), lambda bb: (bb + KSC, 0, 0)),
            pl.BlockSpec(memory_space=pl.ANY),
        ],
        out_specs=pl.BlockSpec((1, F, T), lambda bb: (bb + KSC, 0, 0)),
        out_shape=jax.ShapeDtypeStruct((B, F, T), x.dtype),
        input_output_aliases={2: 0},
    )(sl, x, sc_full)


def kernel(x, seq_lens):
    sl = seq_lens.astype(jnp.int32)
    x2 = x.reshape(ROWS, T)
    sc_full = _sc_part(x2, sl).reshape(B, F, T)
    return _tc_part(sl, x, sc_full)
